# Initial kernel scaffold; baseline (speedup 1.0000x reference)
import jax
import jax.numpy as jnp
from jax.experimental import pallas as pl


def kernel(X, I, lg_kernel, lg_bias, attn_kernel, attn_bias):
    raise NotImplementedError("write your pallas kernel here")


# speedup vs baseline: 7.4729x; 7.4729x over previous
import jax
import jax.numpy as jnp
from jax.experimental import pallas as pl

N_NODES = 50000
F_DIM = 256
CHANNELS = 256
NUM_GRAPHS = 512
ROWS = 1000
NBLOCKS = N_NODES // ROWS


def _fused_kernel(x_ref, i_ref, wl_ref, bl_ref, wa_ref, ba_ref, out_ref):
    step = pl.program_id(0)

    @pl.when(step == 0)
    def _init():
        out_ref[...] = jnp.zeros_like(out_ref)

    x = x_ref[...]
    lin = jnp.dot(x, wl_ref[...], preferred_element_type=jnp.float32) + bl_ref[...]
    att = jnp.dot(x, wa_ref[...], preferred_element_type=jnp.float32) + ba_ref[...]
    masked = lin * jax.nn.sigmoid(att)
    ids = i_ref[0, 0, :]
    seg = jax.lax.broadcasted_iota(jnp.int32, (NUM_GRAPHS, ROWS), 0)
    onehot = (ids[None, :] == seg).astype(jnp.float32)
    out_ref[...] += jnp.dot(onehot, masked, preferred_element_type=jnp.float32)


def kernel(X, I, lg_kernel, lg_bias, attn_kernel, attn_bias):
    ids = I.astype(jnp.int32).reshape(NBLOCKS, 1, ROWS)
    bl = lg_bias.reshape(1, CHANNELS)
    ba = attn_bias.reshape(1, CHANNELS)
    return pl.pallas_call(
        _fused_kernel,
        grid=(NBLOCKS,),
        in_specs=[
            pl.BlockSpec((ROWS, F_DIM), lambda i: (i, 0)),
            pl.BlockSpec((1, 1, ROWS), lambda i: (i, 0, 0)),
            pl.BlockSpec((F_DIM, CHANNELS), lambda i: (0, 0)),
            pl.BlockSpec((1, CHANNELS), lambda i: (0, 0)),
            pl.BlockSpec((F_DIM, CHANNELS), lambda i: (0, 0)),
            pl.BlockSpec((1, CHANNELS), lambda i: (0, 0)),
        ],
        out_specs=pl.BlockSpec((NUM_GRAPHS, CHANNELS), lambda i: (0, 0)),
        out_shape=jax.ShapeDtypeStruct((NUM_GRAPHS, CHANNELS), jnp.float32),
    )(X, ids, lg_kernel, bl, attn_kernel, ba)


# speedup vs baseline: 7.4789x; 1.0008x over previous
import jax
import jax.numpy as jnp
from jax.experimental import pallas as pl

N_NODES = 50000
F_DIM = 256
CHANNELS = 256
NUM_GRAPHS = 512
ROWS = 1000
NBLOCKS = N_NODES // ROWS


def _fused_kernel(x_ref, i_ref, wl_ref, bl_ref, wa_ref, ba_ref, out_ref):
    step = pl.program_id(0)

    @pl.when(step == 0)
    def _init():
        out_ref[...] = jnp.zeros_like(out_ref)

    x = x_ref[...].astype(jnp.bfloat16)
    lin = jnp.dot(x, wl_ref[...].astype(jnp.bfloat16),
                  preferred_element_type=jnp.float32) + bl_ref[...]
    att = jnp.dot(x, wa_ref[...].astype(jnp.bfloat16),
                  preferred_element_type=jnp.float32) + ba_ref[...]
    masked = lin * jax.nn.sigmoid(att)
    ids = i_ref[0, 0, :]
    seg = jax.lax.broadcasted_iota(jnp.int32, (NUM_GRAPHS, ROWS), 0)
    onehot = (ids[None, :] == seg).astype(jnp.bfloat16)
    out_ref[...] += jnp.dot(onehot, masked.astype(jnp.bfloat16),
                            preferred_element_type=jnp.float32)


def kernel(X, I, lg_kernel, lg_bias, attn_kernel, attn_bias):
    ids = I.astype(jnp.int32).reshape(NBLOCKS, 1, ROWS)
    bl = lg_bias.reshape(1, CHANNELS)
    ba = attn_bias.reshape(1, CHANNELS)
    return pl.pallas_call(
        _fused_kernel,
        grid=(NBLOCKS,),
        in_specs=[
            pl.BlockSpec((ROWS, F_DIM), lambda i: (i, 0)),
            pl.BlockSpec((1, 1, ROWS), lambda i: (i, 0, 0)),
            pl.BlockSpec((F_DIM, CHANNELS), lambda i: (0, 0)),
            pl.BlockSpec((1, CHANNELS), lambda i: (0, 0)),
            pl.BlockSpec((F_DIM, CHANNELS), lambda i: (0, 0)),
            pl.BlockSpec((1, CHANNELS), lambda i: (0, 0)),
        ],
        out_specs=pl.BlockSpec((NUM_GRAPHS, CHANNELS), lambda i: (0, 0)),
        out_shape=jax.ShapeDtypeStruct((NUM_GRAPHS, CHANNELS), jnp.float32),
    )(X, ids, lg_kernel, bl, attn_kernel, ba)


# speedup vs baseline: 8.8080x; 1.1777x over previous
import jax
import jax.numpy as jnp
from jax.experimental import pallas as pl

N_NODES = 50000
F_DIM = 256
CHANNELS = 256
NUM_GRAPHS = 512
ROWS = 2000
NBLOCKS = N_NODES // ROWS


def _fused_kernel(x_ref, i_ref, wl_ref, bl_ref, wa_ref, ba_ref, out_ref):
    step = pl.program_id(0)

    @pl.when(step == 0)
    def _init():
        out_ref[...] = jnp.zeros_like(out_ref)

    x = x_ref[...].astype(jnp.bfloat16)
    lin = jnp.dot(x, wl_ref[...].astype(jnp.bfloat16),
                  preferred_element_type=jnp.float32) + bl_ref[...]
    att = jnp.dot(x, wa_ref[...].astype(jnp.bfloat16),
                  preferred_element_type=jnp.float32) + ba_ref[...]
    masked = lin * jax.nn.sigmoid(att)
    ids = i_ref[0, 0, :]
    seg = jax.lax.broadcasted_iota(jnp.int32, (NUM_GRAPHS, ROWS), 0)
    onehot = (ids[None, :] == seg).astype(jnp.bfloat16)
    out_ref[...] += jnp.dot(onehot, masked.astype(jnp.bfloat16),
                            preferred_element_type=jnp.float32)


def kernel(X, I, lg_kernel, lg_bias, attn_kernel, attn_bias):
    ids = I.astype(jnp.int32).reshape(NBLOCKS, 1, ROWS)
    bl = lg_bias.reshape(1, CHANNELS)
    ba = attn_bias.reshape(1, CHANNELS)
    return pl.pallas_call(
        _fused_kernel,
        grid=(NBLOCKS,),
        in_specs=[
            pl.BlockSpec((ROWS, F_DIM), lambda i: (i, 0)),
            pl.BlockSpec((1, 1, ROWS), lambda i: (i, 0, 0)),
            pl.BlockSpec((F_DIM, CHANNELS), lambda i: (0, 0)),
            pl.BlockSpec((1, CHANNELS), lambda i: (0, 0)),
            pl.BlockSpec((F_DIM, CHANNELS), lambda i: (0, 0)),
            pl.BlockSpec((1, CHANNELS), lambda i: (0, 0)),
        ],
        out_specs=pl.BlockSpec((NUM_GRAPHS, CHANNELS), lambda i: (0, 0)),
        out_shape=jax.ShapeDtypeStruct((NUM_GRAPHS, CHANNELS), jnp.float32),
    )(X, ids, lg_kernel, bl, attn_kernel, ba)


# speedup vs baseline: 9.6721x; 1.0981x over previous
import jax
import jax.numpy as jnp
from jax.experimental import pallas as pl

N_NODES = 50000
F_DIM = 256
CHANNELS = 256
NUM_GRAPHS = 512
ROWS = 5000
NBLOCKS = N_NODES // ROWS


def _fused_kernel(x_ref, i_ref, wl_ref, bl_ref, wa_ref, ba_ref, out_ref):
    step = pl.program_id(0)

    @pl.when(step == 0)
    def _init():
        out_ref[...] = jnp.zeros_like(out_ref)

    x = x_ref[...].astype(jnp.bfloat16)
    lin = jnp.dot(x, wl_ref[...].astype(jnp.bfloat16),
                  preferred_element_type=jnp.float32) + bl_ref[...]
    att = jnp.dot(x, wa_ref[...].astype(jnp.bfloat16),
                  preferred_element_type=jnp.float32) + ba_ref[...]
    masked = lin * jax.nn.sigmoid(att)
    ids = i_ref[0, 0, :]
    seg = jax.lax.broadcasted_iota(jnp.int32, (NUM_GRAPHS, ROWS), 0)
    onehot = (ids[None, :] == seg).astype(jnp.bfloat16)
    out_ref[...] += jnp.dot(onehot, masked.astype(jnp.bfloat16),
                            preferred_element_type=jnp.float32)


def kernel(X, I, lg_kernel, lg_bias, attn_kernel, attn_bias):
    ids = I.astype(jnp.int32).reshape(NBLOCKS, 1, ROWS)
    bl = lg_bias.reshape(1, CHANNELS)
    ba = attn_bias.reshape(1, CHANNELS)
    return pl.pallas_call(
        _fused_kernel,
        grid=(NBLOCKS,),
        in_specs=[
            pl.BlockSpec((ROWS, F_DIM), lambda i: (i, 0)),
            pl.BlockSpec((1, 1, ROWS), lambda i: (i, 0, 0)),
            pl.BlockSpec((F_DIM, CHANNELS), lambda i: (0, 0)),
            pl.BlockSpec((1, CHANNELS), lambda i: (0, 0)),
            pl.BlockSpec((F_DIM, CHANNELS), lambda i: (0, 0)),
            pl.BlockSpec((1, CHANNELS), lambda i: (0, 0)),
        ],
        out_specs=pl.BlockSpec((NUM_GRAPHS, CHANNELS), lambda i: (0, 0)),
        out_shape=jax.ShapeDtypeStruct((NUM_GRAPHS, CHANNELS), jnp.float32),
    )(X, ids, lg_kernel, bl, attn_kernel, ba)


# speedup vs baseline: 9.7996x; 1.0132x over previous
import jax
import jax.numpy as jnp
from jax.experimental import pallas as pl

N_NODES = 50000
F_DIM = 256
CHANNELS = 256
NUM_GRAPHS = 512
ROWS = 10000
NBLOCKS = N_NODES // ROWS


def _fused_kernel(x_ref, i_ref, wl_ref, bl_ref, wa_ref, ba_ref, out_ref):
    step = pl.program_id(0)

    @pl.when(step == 0)
    def _init():
        out_ref[...] = jnp.zeros_like(out_ref)

    x = x_ref[...].astype(jnp.bfloat16)
    lin = jnp.dot(x, wl_ref[...].astype(jnp.bfloat16),
                  preferred_element_type=jnp.float32) + bl_ref[...]
    att = jnp.dot(x, wa_ref[...].astype(jnp.bfloat16),
                  preferred_element_type=jnp.float32) + ba_ref[...]
    masked = lin * jax.nn.sigmoid(att)
    ids = i_ref[0, 0, :]
    seg = jax.lax.broadcasted_iota(jnp.int32, (NUM_GRAPHS, ROWS), 0)
    onehot = (ids[None, :] == seg).astype(jnp.bfloat16)
    out_ref[...] += jnp.dot(onehot, masked.astype(jnp.bfloat16),
                            preferred_element_type=jnp.float32)


def kernel(X, I, lg_kernel, lg_bias, attn_kernel, attn_bias):
    ids = I.astype(jnp.int32).reshape(NBLOCKS, 1, ROWS)
    bl = lg_bias.reshape(1, CHANNELS)
    ba = attn_bias.reshape(1, CHANNELS)
    return pl.pallas_call(
        _fused_kernel,
        grid=(NBLOCKS,),
        in_specs=[
            pl.BlockSpec((ROWS, F_DIM), lambda i: (i, 0)),
            pl.BlockSpec((1, 1, ROWS), lambda i: (i, 0, 0)),
            pl.BlockSpec((F_DIM, CHANNELS), lambda i: (0, 0)),
            pl.BlockSpec((1, CHANNELS), lambda i: (0, 0)),
            pl.BlockSpec((F_DIM, CHANNELS), lambda i: (0, 0)),
            pl.BlockSpec((1, CHANNELS), lambda i: (0, 0)),
        ],
        out_specs=pl.BlockSpec((NUM_GRAPHS, CHANNELS), lambda i: (0, 0)),
        out_shape=jax.ShapeDtypeStruct((NUM_GRAPHS, CHANNELS), jnp.float32),
    )(X, ids, lg_kernel, bl, attn_kernel, ba)
